# Initial kernel scaffold; baseline (speedup 1.0000x reference)
import jax
import jax.numpy as jnp
from jax.experimental import pallas as pl


def kernel(gt_bboxes, images, anchors):
    raise NotImplementedError("write your pallas kernel here")


# speedup vs baseline: 49.0883x; 49.0883x over previous
import jax
import jax.numpy as jnp
from jax import lax
from jax.experimental import pallas as pl
from jax.experimental.pallas import tpu as pltpu

_TOPK = 9


def _assign_body(gt_ref, anch_ref, pos_ref, neg_ref):
    ax1 = anch_ref[0:1, :]
    ay1 = anch_ref[1:2, :]
    ax2 = anch_ref[2:3, :]
    ay2 = anch_ref[3:4, :]
    acx = (ax1 + ax2) / 2
    acy = (ay1 + ay2) / 2

    gt = gt_ref[0]
    gx1 = gt[:, 0:1]
    gy1 = gt[:, 1:2]
    gx2 = gt[:, 2:3]
    gy2 = gt[:, 3:4]
    gcx = (gx1 + gx2) / 2
    gcy = (gy1 + gy2) / 2

    m = gt.shape[0]
    ap = anch_ref.shape[1]
    a = pos_ref.shape[2]

    dx = gcx - acx
    dy = gcy - acy
    d = jnp.sqrt(dx * dx + dy * dy)

    ox = jnp.minimum(gx2, ax2) - jnp.maximum(gx1, ax1)
    oy = jnp.minimum(gy2, ay2) - jnp.maximum(gy1, ay1)
    overlap = jnp.maximum(ox, 0.0) * jnp.maximum(oy, 0.0)
    area1 = jnp.maximum(gx2 - gx1, 0.0) * jnp.maximum(gy2 - gy1, 0.0)
    area2 = jnp.maximum(ax2 - ax1, 0.0) * jnp.maximum(ay2 - ay1, 0.0)
    union = area1 + area2 - overlap + 1e-9
    iou = overlap / union

    iota = lax.broadcasted_iota(jnp.int32, (m, ap), 1)
    vals = []
    dcur = d
    for _ in range(_TOPK):
        mv = jnp.min(dcur, axis=1, keepdims=True)
        idx = jnp.min(jnp.where(dcur == mv, iota, ap), axis=1, keepdims=True)
        onehot = iota == idx
        vals.append(jnp.sum(jnp.where(onehot, iou, 0.0), axis=1, keepdims=True))
        dcur = jnp.where(onehot, jnp.float32(jnp.inf), dcur)
    tv = jnp.concatenate(vals, axis=1)

    msk = (tv > 0).astype(jnp.float32)
    n = jnp.sum(msk)
    mean = jnp.sum(tv * msk) / n
    var = jnp.sum(((tv - mean) ** 2) * msk) / (n - 1.0)
    thr = mean + jnp.sqrt(var)

    iou_v = iou[:, :a]
    pos_ref[0] = iou_v > thr
    neg_ref[0] = iou_v < thr


def kernel(gt_bboxes, images, anchors):
    del images
    b, m, _ = gt_bboxes.shape
    a = anchors.shape[0]
    ap = ((a + 127) // 128) * 128

    anch = jnp.concatenate(
        [anchors, jnp.full((ap - a, 4), 1e9, jnp.float32)], axis=0
    )
    anch_t = jnp.concatenate([anch.T, jnp.zeros((4, ap), jnp.float32)], axis=0)

    pos, neg = pl.pallas_call(
        _assign_body,
        grid=(b,),
        in_specs=[
            pl.BlockSpec((1, m, 4), lambda i: (i, 0, 0)),
            pl.BlockSpec((8, ap), lambda i: (0, 0)),
        ],
        out_specs=[
            pl.BlockSpec((1, m, a), lambda i: (i, 0, 0)),
            pl.BlockSpec((1, m, a), lambda i: (i, 0, 0)),
        ],
        out_shape=[
            jax.ShapeDtypeStruct((b, m, a), jnp.bool_),
            jax.ShapeDtypeStruct((b, m, a), jnp.bool_),
        ],
    )(gt_bboxes, anch_t)

    assigned_scores = jnp.ones((b, a), jnp.float32)
    return pos, neg, assigned_scores
